# baseline (speedup 1.0000x reference)
import functools

import jax
import jax.numpy as jnp
from jax import lax
from jax.experimental import pallas as pl
from jax.experimental.pallas import tpu as pltpu
from jax.experimental.pallas import tpu_sc as plsc

N = 10000
NP = 10240
E = 320000
D = 128

NC = 2
NS = 16
NW = NC * NS

E_TILE = E // NW
CHUNK = 128
NCHUNK = 79
E_TILE_P = NCHUNK * CHUNK
N_PAIR = NCHUNK // 2

ROWS_SUB = NP // NS

_mesh = plsc.VectorSubcoreMesh(core_axis_name="c", subcore_axis_name="s")


@functools.partial(
    pl.kernel,
    out_type=jax.ShapeDtypeStruct((NW, NP), jnp.float32),
    mesh=_mesh,
    compiler_params=pltpu.CompilerParams(needs_layout_passes=False),
    scratch_types=[
        pltpu.VMEM((NCHUNK, CHUNK), jnp.int32),
        pltpu.VMEM((NP,), jnp.float32),
    ],
)
def _deg_sc(row_hbm, out_hbm, idx_all, hist_v):
    c = lax.axis_index("c")
    s = lax.axis_index("s")
    wid = s * NC + c

    pltpu.sync_copy(row_hbm.at[wid], idx_all)

    zeros16 = jnp.zeros((16,), jnp.float32)

    def zbody(i, carry):
        hist_v[pl.ds(i * 16, 16)] = zeros16
        return carry

    lax.fori_loop(0, NP // 16, zbody, 0)

    ones16 = jnp.ones((16,), jnp.float32)

    def cbody(j, carry):
        for k in range(CHUNK // 16):
            idx16 = idx_all[j, pl.ds(k * 16, 16)]
            plsc.addupdate_scatter(hist_v, [idx16], ones16)
        return carry

    lax.fori_loop(0, NCHUNK, cbody, 0)

    pltpu.sync_copy(hist_v, out_hbm.at[wid])


N_QUAD = 19


@functools.partial(
    pl.kernel,
    out_type=jax.ShapeDtypeStruct((NC, NP, D), jnp.float32),
    mesh=_mesh,
    scratch_types=[
        pltpu.VMEM_SHARED((NP, D), jnp.float32),
        pltpu.VMEM((2, CHUNK), jnp.int32),
        pltpu.VMEM((2, CHUNK), jnp.int32),
        pltpu.VMEM((2, CHUNK), jnp.int32),
        pltpu.VMEM((2, CHUNK), jnp.int32),
        pltpu.VMEM((CHUNK, D), jnp.float32),
        pltpu.VMEM((CHUNK, D), jnp.float32),
        pltpu.SemaphoreType.DMA,
        pltpu.SemaphoreType.DMA,
        pltpu.SemaphoreType.DMA,
        pltpu.SemaphoreType.DMA,
        pltpu.SemaphoreType.DMA,
        pltpu.SemaphoreType.DMA,
        pltpu.SemaphoreType.DMA,
        pltpu.SemaphoreType.DMA,
        pltpu.SemaphoreType.DMA,
    ],
)
def _scatter_sc(hp_hbm, rc_hbm, zeros_hbm, out_hbm,
                acc_sh, rc0, rc1, rc2, rc3, msg0, msg1,
                gsem0, gsem1, ssem0, ssem1,
                isem0, isem1, isem2, isem3, lsem):
    c = lax.axis_index("c")
    s = lax.axis_index("s")
    wid = s * NC + c
    r0 = s * ROWS_SUB
    cbase = wid * NCHUNK

    rcs = [rc0, rc1, rc2, rc3]
    isems = [isem0, isem1, isem2, isem3]
    msgs = [msg0, msg1]
    gsems = [gsem0, gsem1]
    ssems = [ssem0, ssem1]

    def idx_load(j, k):
        pltpu.async_copy(rc_hbm.at[cbase + j], rcs[k], isems[k])

    def idx_wait(j, k):
        pltpu.make_async_copy(rc_hbm.at[cbase + j], rcs[k], isems[k]).wait()

    def gather(k, m):
        pltpu.async_copy(hp_hbm.at[rcs[k].at[0]], msgs[m], gsems[m])

    def gather_wait(k, m):
        pltpu.make_async_copy(hp_hbm.at[rcs[k].at[0]], msgs[m], gsems[m]).wait()

    def scatter(k, m):
        pltpu.async_copy(msgs[m], acc_sh.at[rcs[k].at[1]], ssems[m], add=True)

    def scatter_wait(k, m):
        pltpu.make_async_copy(msgs[m], acc_sh.at[rcs[k].at[1]], ssems[m]).wait()

    def step(cc, phase, prefetch=True):
        k, m = phase % 4, phase % 2
        idx_wait(cc + 1, (phase + 1) % 4)
        gather_wait(k, m)
        scatter(k, m)
        scatter_wait((phase - 1) % 4, 1 - m)
        gather((phase + 1) % 4, 1 - m)
        if prefetch:
            idx_load(jnp.minimum(cc + 3, NCHUNK - 1), (phase + 3) % 4)

    zld = pltpu.async_copy(zeros_hbm.at[pl.ds(r0, ROWS_SUB)],
                           acc_sh.at[pl.ds(r0, ROWS_SUB)], lsem)
    idx_load(0, 0)
    idx_load(1, 1)
    idx_load(2, 2)
    idx_wait(0, 0)
    gather(0, 0)
    zld.wait()
    plsc.subcore_barrier()

    idx_wait(1, 1)
    gather_wait(0, 0)
    scatter(0, 0)
    gather(1, 1)
    idx_load(3, 3)

    def quad_body(q, carry):
        base = 4 * q + 1
        for sph in range(4):
            step(base + sph, 1 + sph)
        return carry

    lax.fori_loop(0, N_QUAD, quad_body, 0)

    idx_wait(NCHUNK - 1, 2)
    gather_wait(1, 1)
    scatter(1, 1)
    scatter_wait(0, 0)
    gather(2, 0)
    idx_wait(NCHUNK - 1, 3)
    gather_wait(2, 0)
    scatter(2, 0)
    scatter_wait(1, 1)
    scatter_wait(2, 0)

    plsc.subcore_barrier()
    pltpu.sync_copy(acc_sh.at[pl.ds(r0, ROWS_SUB)], out_hbm.at[c, pl.ds(r0, ROWS_SUB)])


BN = 1024
GRID = NP // BN


def _dis_from_hist(hist_blk):
    deg = jnp.sum(hist_blk, axis=0) + 1.0
    return lax.rsqrt(deg)


def _pre_body(hist_ref, x_ref, w1_ref, b1_ref, out_ref):
    dis = _dis_from_hist(hist_ref[...])
    h = lax.dot_general(x_ref[...], w1_ref[...], (((1,), (1,)), ((), ())),
                        preferred_element_type=jnp.float32) + b1_ref[...]
    out_ref[...] = dis[:, None] * h


_pre_tc = pl.pallas_call(
    _pre_body,
    grid=(GRID,),
    in_specs=[
        pl.BlockSpec((NW, BN), lambda i: (0, i)),
        pl.BlockSpec((BN, D), lambda i: (i, 0)),
        pl.BlockSpec((D, D), lambda i: (0, 0)),
        pl.BlockSpec((1, D), lambda i: (0, 0)),
    ],
    out_specs=pl.BlockSpec((BN, D), lambda i: (i, 0)),
    out_shape=jax.ShapeDtypeStruct((NP, D), jnp.float32),
)


def _mid_body(hist_ref, p_ref, hp_ref, w2_ref, b2_ref, out_ref):
    dis = _dis_from_hist(hist_ref[...])[:, None]
    sacc = p_ref[0] + p_ref[1] + hp_ref[...]
    y = jnp.maximum(dis * sacc, 0.0)
    h2 = lax.dot_general(y, w2_ref[...], (((1,), (1,)), ((), ())),
                         preferred_element_type=jnp.float32) + b2_ref[...]
    out_ref[...] = dis * h2


_mid_tc = pl.pallas_call(
    _mid_body,
    grid=(GRID,),
    in_specs=[
        pl.BlockSpec((NW, BN), lambda i: (0, i)),
        pl.BlockSpec((NC, BN, D), lambda i: (0, i, 0)),
        pl.BlockSpec((BN, D), lambda i: (i, 0)),
        pl.BlockSpec((D, D), lambda i: (0, 0)),
        pl.BlockSpec((1, D), lambda i: (0, 0)),
    ],
    out_specs=pl.BlockSpec((BN, D), lambda i: (i, 0)),
    out_shape=jax.ShapeDtypeStruct((NP, D), jnp.float32),
)


def _post_body(hist_ref, p_ref, hp_ref, out_ref):
    dis = _dis_from_hist(hist_ref[...])[:, None]
    o = dis * (p_ref[0] + p_ref[1] + hp_ref[...])
    m = jnp.max(o, axis=1, keepdims=True)
    e = o - m
    out_ref[...] = e - jnp.log(jnp.sum(jnp.exp(e), axis=1, keepdims=True))


_post_tc = pl.pallas_call(
    _post_body,
    grid=(GRID,),
    in_specs=[
        pl.BlockSpec((NW, BN), lambda i: (0, i)),
        pl.BlockSpec((NC, BN, D), lambda i: (0, i, 0)),
        pl.BlockSpec((BN, D), lambda i: (i, 0)),
    ],
    out_specs=pl.BlockSpec((BN, D), lambda i: (i, 0)),
    out_shape=jax.ShapeDtypeStruct((NP, D), jnp.float32),
)


def _pad_edges(v):
    vt = v.reshape(NW, E_TILE)
    padv = N + (jnp.arange(E_TILE_P - E_TILE, dtype=jnp.int32) % (NP - N))
    padv = jnp.broadcast_to(padv, (NW, E_TILE_P - E_TILE))
    return jnp.concatenate([vt, padv], axis=1).reshape(NW, NCHUNK, CHUNK)


def kernel(x, edge_index, W1, b1, W2, b2):
    row = _pad_edges(edge_index[0])
    col = _pad_edges(edge_index[1])
    rc = jnp.stack([row, col], axis=2).reshape(NW * NCHUNK, 2, CHUNK)
    xp = jnp.pad(x, ((0, NP - N), (0, 0)))
    zeros = jnp.zeros((NP, D), jnp.float32)
    hist = _deg_sc(row)
    h1p = _pre_tc(hist, xp, W1, b1.reshape(1, D))
    p1 = _scatter_sc(h1p, rc, zeros)
    h2p = _mid_tc(hist, p1, h1p, W2, b2.reshape(1, D))
    p2 = _scatter_sc(h2p, rc, zeros)
    return _post_tc(hist, p2, h2p)[:N]


# speedup vs baseline: 1.1598x; 1.1598x over previous
import functools

import jax
import jax.numpy as jnp
from jax import lax
from jax.experimental import pallas as pl
from jax.experimental.pallas import tpu as pltpu
from jax.experimental.pallas import tpu_sc as plsc

N = 10000
NP = 10240
E = 320000
D = 128

NC = 2
NS = 16
NW = NC * NS

E_TILE = E // NW
CHUNK = 128
NCHUNK = 79
E_TILE_P = NCHUNK * CHUNK
N_PAIR = NCHUNK // 2

ROWS_SUB = NP // NS

_mesh = plsc.VectorSubcoreMesh(core_axis_name="c", subcore_axis_name="s")


@functools.partial(
    pl.kernel,
    out_type=jax.ShapeDtypeStruct((NW, NP), jnp.float32),
    mesh=_mesh,
    compiler_params=pltpu.CompilerParams(needs_layout_passes=False),
    scratch_types=[
        pltpu.VMEM((NCHUNK, CHUNK), jnp.int32),
        pltpu.VMEM((NP,), jnp.float32),
    ],
)
def _deg_sc(row_hbm, out_hbm, idx_all, hist_v):
    c = lax.axis_index("c")
    s = lax.axis_index("s")
    wid = s * NC + c

    pltpu.sync_copy(row_hbm.at[wid], idx_all)

    zeros16 = jnp.zeros((16,), jnp.float32)

    def zbody(i, carry):
        hist_v[pl.ds(i * 16, 16)] = zeros16
        return carry

    lax.fori_loop(0, NP // 16, zbody, 0)

    ones16 = jnp.ones((16,), jnp.float32)

    def cbody(j, carry):
        for k in range(CHUNK // 16):
            idx16 = idx_all[j, pl.ds(k * 16, 16)]
            plsc.addupdate_scatter(hist_v, [idx16], ones16)
        return carry

    lax.fori_loop(0, NCHUNK, cbody, 0)

    pltpu.sync_copy(hist_v, out_hbm.at[wid])


N_QUAD = 19


@functools.partial(
    pl.kernel,
    out_type=jax.ShapeDtypeStruct((NC, NP, D), jnp.float32),
    mesh=_mesh,
    scratch_types=[
        pltpu.VMEM_SHARED((NP, D), jnp.float32),
        pltpu.VMEM((2, CHUNK), jnp.int32),
        pltpu.VMEM((2, CHUNK), jnp.int32),
        pltpu.VMEM((2, CHUNK), jnp.int32),
        pltpu.VMEM((2, CHUNK), jnp.int32),
        pltpu.VMEM((CHUNK, D), jnp.float32),
        pltpu.VMEM((CHUNK, D), jnp.float32),
        pltpu.SemaphoreType.DMA,
        pltpu.SemaphoreType.DMA,
        pltpu.SemaphoreType.DMA,
        pltpu.SemaphoreType.DMA,
        pltpu.SemaphoreType.DMA,
        pltpu.SemaphoreType.DMA,
        pltpu.SemaphoreType.DMA,
        pltpu.SemaphoreType.DMA,
        pltpu.SemaphoreType.DMA,
    ],
)
def _scatter_sc(hp_hbm, rc_hbm, zeros_hbm, out_hbm,
                acc_sh, rc0, rc1, rc2, rc3, msg0, msg1,
                gsem0, gsem1, ssem0, ssem1,
                isem0, isem1, isem2, isem3, lsem):
    c = lax.axis_index("c")
    s = lax.axis_index("s")
    wid = s * NC + c
    r0 = s * ROWS_SUB
    cbase = wid * NCHUNK

    rcs = [rc0, rc1, rc2, rc3]
    isems = [isem0, isem1, isem2, isem3]
    msgs = [msg0, msg1]
    gsems = [gsem0, gsem1]
    ssems = [ssem0, ssem1]

    def idx_load(j, k):
        pltpu.async_copy(rc_hbm.at[cbase + j], rcs[k], isems[k])

    def idx_wait(j, k):
        pltpu.make_async_copy(rc_hbm.at[cbase + j], rcs[k], isems[k]).wait()

    def gather(k, m):
        pltpu.async_copy(hp_hbm.at[rcs[k].at[0]], msgs[m], gsems[m])

    def gather_wait(k, m):
        pltpu.make_async_copy(hp_hbm.at[rcs[k].at[0]], msgs[m], gsems[m]).wait()

    def scatter(k, m):
        pltpu.async_copy(msgs[m], acc_sh.at[rcs[k].at[1]], ssems[m], add=True)

    def scatter_wait(k, m):
        pltpu.make_async_copy(msgs[m], acc_sh.at[rcs[k].at[1]], ssems[m]).wait()

    def step(cc, phase, prefetch=True):
        k, m = phase % 4, phase % 2
        if prefetch:
            idx_wait(cc + 1, (phase + 1) % 4)
            gather((phase + 1) % 4, 1 - m)
        gather_wait(k, m)
        pltpu.sync_copy(msgs[m], acc_sh.at[rcs[k].at[1]], add=True)
        if prefetch:
            idx_load(cc + 3, (phase + 3) % 4)

    zld = pltpu.async_copy(zeros_hbm.at[pl.ds(r0, ROWS_SUB)],
                           acc_sh.at[pl.ds(r0, ROWS_SUB)], lsem)
    idx_load(0, 0)
    idx_load(1, 1)
    idx_load(2, 2)
    idx_wait(0, 0)
    gather(0, 0)
    zld.wait()
    plsc.subcore_barrier()

    def quad_body(q, carry):
        base = 4 * q
        for sph in range(4):
            step(base + sph, sph)
        return carry

    lax.fori_loop(0, N_QUAD, quad_body, 0)

    idx_wait(NCHUNK - 2, 1)
    gather(1, 1)
    gather_wait(0, 0)
    pltpu.sync_copy(msg0, acc_sh.at[rc0.at[1]], add=True)
    idx_wait(NCHUNK - 1, 2)
    gather(2, 0)
    gather_wait(1, 1)
    pltpu.sync_copy(msg1, acc_sh.at[rc1.at[1]], add=True)
    gather_wait(2, 0)
    pltpu.sync_copy(msg0, acc_sh.at[rc2.at[1]], add=True)

    plsc.subcore_barrier()
    pltpu.sync_copy(acc_sh.at[pl.ds(r0, ROWS_SUB)], out_hbm.at[c, pl.ds(r0, ROWS_SUB)])


BN = 1024
GRID = NP // BN


def _dis_from_hist(hist_blk):
    deg = jnp.sum(hist_blk, axis=0) + 1.0
    return lax.rsqrt(deg)


def _pre_body(hist_ref, x_ref, w1_ref, b1_ref, out_ref):
    dis = _dis_from_hist(hist_ref[...])
    h = lax.dot_general(x_ref[...], w1_ref[...], (((1,), (1,)), ((), ())),
                        preferred_element_type=jnp.float32) + b1_ref[...]
    out_ref[...] = dis[:, None] * h


_pre_tc = pl.pallas_call(
    _pre_body,
    grid=(GRID,),
    in_specs=[
        pl.BlockSpec((NW, BN), lambda i: (0, i)),
        pl.BlockSpec((BN, D), lambda i: (i, 0)),
        pl.BlockSpec((D, D), lambda i: (0, 0)),
        pl.BlockSpec((1, D), lambda i: (0, 0)),
    ],
    out_specs=pl.BlockSpec((BN, D), lambda i: (i, 0)),
    out_shape=jax.ShapeDtypeStruct((NP, D), jnp.float32),
)


def _mid_body(hist_ref, p_ref, hp_ref, w2_ref, b2_ref, out_ref):
    dis = _dis_from_hist(hist_ref[...])[:, None]
    sacc = p_ref[0] + p_ref[1] + hp_ref[...]
    y = jnp.maximum(dis * sacc, 0.0)
    h2 = lax.dot_general(y, w2_ref[...], (((1,), (1,)), ((), ())),
                         preferred_element_type=jnp.float32) + b2_ref[...]
    out_ref[...] = dis * h2


_mid_tc = pl.pallas_call(
    _mid_body,
    grid=(GRID,),
    in_specs=[
        pl.BlockSpec((NW, BN), lambda i: (0, i)),
        pl.BlockSpec((NC, BN, D), lambda i: (0, i, 0)),
        pl.BlockSpec((BN, D), lambda i: (i, 0)),
        pl.BlockSpec((D, D), lambda i: (0, 0)),
        pl.BlockSpec((1, D), lambda i: (0, 0)),
    ],
    out_specs=pl.BlockSpec((BN, D), lambda i: (i, 0)),
    out_shape=jax.ShapeDtypeStruct((NP, D), jnp.float32),
)


def _post_body(hist_ref, p_ref, hp_ref, out_ref):
    dis = _dis_from_hist(hist_ref[...])[:, None]
    o = dis * (p_ref[0] + p_ref[1] + hp_ref[...])
    m = jnp.max(o, axis=1, keepdims=True)
    e = o - m
    out_ref[...] = e - jnp.log(jnp.sum(jnp.exp(e), axis=1, keepdims=True))


_post_tc = pl.pallas_call(
    _post_body,
    grid=(GRID,),
    in_specs=[
        pl.BlockSpec((NW, BN), lambda i: (0, i)),
        pl.BlockSpec((NC, BN, D), lambda i: (0, i, 0)),
        pl.BlockSpec((BN, D), lambda i: (i, 0)),
    ],
    out_specs=pl.BlockSpec((BN, D), lambda i: (i, 0)),
    out_shape=jax.ShapeDtypeStruct((NP, D), jnp.float32),
)


def _pad_edges(v):
    vt = v.reshape(NW, E_TILE)
    padv = N + (jnp.arange(E_TILE_P - E_TILE, dtype=jnp.int32) % (NP - N))
    padv = jnp.broadcast_to(padv, (NW, E_TILE_P - E_TILE))
    return jnp.concatenate([vt, padv], axis=1).reshape(NW, NCHUNK, CHUNK)


def kernel(x, edge_index, W1, b1, W2, b2):
    row = _pad_edges(edge_index[0])
    col = _pad_edges(edge_index[1])
    rc = jnp.stack([row, col], axis=2).reshape(NW * NCHUNK, 2, CHUNK)
    xp = jnp.pad(x, ((0, NP - N), (0, 0)))
    zeros = jnp.zeros((NP, D), jnp.float32)
    hist = _deg_sc(row)
    h1p = _pre_tc(hist, xp, W1, b1.reshape(1, D))
    p1 = _scatter_sc(h1p, rc, zeros)
    h2p = _mid_tc(hist, p1, h1p, W2, b2.reshape(1, D))
    p2 = _scatter_sc(h2p, rc, zeros)
    return _post_tc(hist, p2, h2p)[:N]


# speedup vs baseline: 1.1691x; 1.0080x over previous
import functools

import jax
import jax.numpy as jnp
from jax import lax
from jax.experimental import pallas as pl
from jax.experimental.pallas import tpu as pltpu
from jax.experimental.pallas import tpu_sc as plsc

N = 10000
NP = 10240
E = 320000
D = 128

NC = 2
NS = 16
NW = NC * NS

E_TILE = E // NW
CHUNK = 128
NCHUNK = 79
E_TILE_P = NCHUNK * CHUNK
N_PAIR = NCHUNK // 2

ROWS_SUB = NP // NS

_mesh = plsc.VectorSubcoreMesh(core_axis_name="c", subcore_axis_name="s")


@functools.partial(
    pl.kernel,
    out_type=jax.ShapeDtypeStruct((NW, NP), jnp.float32),
    mesh=_mesh,
    compiler_params=pltpu.CompilerParams(needs_layout_passes=False),
    scratch_types=[
        pltpu.VMEM((NCHUNK, CHUNK), jnp.int32),
        pltpu.VMEM((NP,), jnp.float32),
        pltpu.SemaphoreType.DMA,
        pltpu.SemaphoreType.DMA,
    ],
)
def _deg_sc(row_hbm, zeros_hbm, out_hbm, idx_all, hist_v, isem, zsem):
    c = lax.axis_index("c")
    s = lax.axis_index("s")
    wid = s * NC + c

    ild = pltpu.async_copy(row_hbm.at[wid], idx_all, isem)
    zld = pltpu.async_copy(zeros_hbm, hist_v, zsem)
    ild.wait()
    zld.wait()

    ones16 = jnp.ones((16,), jnp.float32)

    def cbody(j, carry):
        for k in range(CHUNK // 16):
            idx16 = idx_all[j, pl.ds(k * 16, 16)]
            plsc.addupdate_scatter(hist_v, [idx16], ones16)
        return carry

    lax.fori_loop(0, NCHUNK, cbody, 0)

    pltpu.sync_copy(hist_v, out_hbm.at[wid])


N_QUAD = 19


@functools.partial(
    pl.kernel,
    out_type=jax.ShapeDtypeStruct((NC, NP, D), jnp.float32),
    mesh=_mesh,
    scratch_types=[
        pltpu.VMEM_SHARED((NP, D), jnp.float32),
        pltpu.VMEM((2, CHUNK), jnp.int32),
        pltpu.VMEM((2, CHUNK), jnp.int32),
        pltpu.VMEM((2, CHUNK), jnp.int32),
        pltpu.VMEM((2, CHUNK), jnp.int32),
        pltpu.VMEM((CHUNK, D), jnp.float32),
        pltpu.VMEM((CHUNK, D), jnp.float32),
        pltpu.SemaphoreType.DMA,
        pltpu.SemaphoreType.DMA,
        pltpu.SemaphoreType.DMA,
        pltpu.SemaphoreType.DMA,
        pltpu.SemaphoreType.DMA,
        pltpu.SemaphoreType.DMA,
        pltpu.SemaphoreType.DMA,
        pltpu.SemaphoreType.DMA,
        pltpu.SemaphoreType.DMA,
    ],
)
def _scatter_sc(hp_hbm, rc_hbm, zeros_hbm, out_hbm,
                acc_sh, rc0, rc1, rc2, rc3, msg0, msg1,
                gsem0, gsem1, ssem0, ssem1,
                isem0, isem1, isem2, isem3, lsem):
    c = lax.axis_index("c")
    s = lax.axis_index("s")
    wid = s * NC + c
    r0 = s * ROWS_SUB
    cbase = wid * NCHUNK

    rcs = [rc0, rc1, rc2, rc3]
    isems = [isem0, isem1, isem2, isem3]
    msgs = [msg0, msg1]
    gsems = [gsem0, gsem1]
    ssems = [ssem0, ssem1]

    def idx_load(j, k):
        pltpu.async_copy(rc_hbm.at[cbase + j], rcs[k], isems[k])

    def idx_wait(j, k):
        pltpu.make_async_copy(rc_hbm.at[cbase + j], rcs[k], isems[k]).wait()

    def gather(k, m):
        pltpu.async_copy(hp_hbm.at[rcs[k].at[0]], msgs[m], gsems[m])

    def gather_wait(k, m):
        pltpu.make_async_copy(hp_hbm.at[rcs[k].at[0]], msgs[m], gsems[m]).wait()

    def scatter(k, m):
        pltpu.async_copy(msgs[m], acc_sh.at[rcs[k].at[1]], ssems[m], add=True)

    def scatter_wait(k, m):
        pltpu.make_async_copy(msgs[m], acc_sh.at[rcs[k].at[1]], ssems[m]).wait()

    def step(cc, phase, prefetch=True):
        k, m = phase % 4, phase % 2
        if prefetch:
            idx_wait(cc + 1, (phase + 1) % 4)
            gather((phase + 1) % 4, 1 - m)
        gather_wait(k, m)
        pltpu.sync_copy(msgs[m], acc_sh.at[rcs[k].at[1]], add=True)
        if prefetch:
            idx_load(cc + 3, (phase + 3) % 4)

    zld = pltpu.async_copy(zeros_hbm.at[pl.ds(r0, ROWS_SUB)],
                           acc_sh.at[pl.ds(r0, ROWS_SUB)], lsem)
    idx_load(0, 0)
    idx_load(1, 1)
    idx_load(2, 2)
    idx_wait(0, 0)
    gather(0, 0)
    zld.wait()
    plsc.subcore_barrier()

    def quad_body(q, carry):
        base = 4 * q
        for sph in range(4):
            step(base + sph, sph)
        return carry

    lax.fori_loop(0, N_QUAD, quad_body, 0)

    idx_wait(NCHUNK - 2, 1)
    gather(1, 1)
    gather_wait(0, 0)
    pltpu.sync_copy(msg0, acc_sh.at[rc0.at[1]], add=True)
    idx_wait(NCHUNK - 1, 2)
    gather(2, 0)
    gather_wait(1, 1)
    pltpu.sync_copy(msg1, acc_sh.at[rc1.at[1]], add=True)
    gather_wait(2, 0)
    pltpu.sync_copy(msg0, acc_sh.at[rc2.at[1]], add=True)

    plsc.subcore_barrier()
    pltpu.sync_copy(acc_sh.at[pl.ds(r0, ROWS_SUB)], out_hbm.at[c, pl.ds(r0, ROWS_SUB)])


BN = 2048
GRID = NP // BN


def _dis_from_hist(hist_blk):
    deg = jnp.sum(hist_blk, axis=0) + 1.0
    return lax.rsqrt(deg)


def _pre_body(hist_ref, x_ref, w1_ref, b1_ref, out_ref):
    dis = _dis_from_hist(hist_ref[...])
    h = lax.dot_general(x_ref[...], w1_ref[...], (((1,), (1,)), ((), ())),
                        preferred_element_type=jnp.float32) + b1_ref[...]
    out_ref[...] = dis[:, None] * h


_pre_tc = pl.pallas_call(
    _pre_body,
    grid=(GRID,),
    in_specs=[
        pl.BlockSpec((NW, BN), lambda i: (0, i)),
        pl.BlockSpec((BN, D), lambda i: (i, 0)),
        pl.BlockSpec((D, D), lambda i: (0, 0)),
        pl.BlockSpec((1, D), lambda i: (0, 0)),
    ],
    out_specs=pl.BlockSpec((BN, D), lambda i: (i, 0)),
    out_shape=jax.ShapeDtypeStruct((NP, D), jnp.float32),
)


def _mid_body(hist_ref, p_ref, hp_ref, w2_ref, b2_ref, out_ref):
    dis = _dis_from_hist(hist_ref[...])[:, None]
    sacc = p_ref[0] + p_ref[1] + hp_ref[...]
    y = jnp.maximum(dis * sacc, 0.0)
    h2 = lax.dot_general(y, w2_ref[...], (((1,), (1,)), ((), ())),
                         preferred_element_type=jnp.float32) + b2_ref[...]
    out_ref[...] = dis * h2


_mid_tc = pl.pallas_call(
    _mid_body,
    grid=(GRID,),
    in_specs=[
        pl.BlockSpec((NW, BN), lambda i: (0, i)),
        pl.BlockSpec((NC, BN, D), lambda i: (0, i, 0)),
        pl.BlockSpec((BN, D), lambda i: (i, 0)),
        pl.BlockSpec((D, D), lambda i: (0, 0)),
        pl.BlockSpec((1, D), lambda i: (0, 0)),
    ],
    out_specs=pl.BlockSpec((BN, D), lambda i: (i, 0)),
    out_shape=jax.ShapeDtypeStruct((NP, D), jnp.float32),
)


def _post_body(hist_ref, p_ref, hp_ref, out_ref):
    dis = _dis_from_hist(hist_ref[...])[:, None]
    o = dis * (p_ref[0] + p_ref[1] + hp_ref[...])
    m = jnp.max(o, axis=1, keepdims=True)
    e = o - m
    out_ref[...] = e - jnp.log(jnp.sum(jnp.exp(e), axis=1, keepdims=True))


_post_tc = pl.pallas_call(
    _post_body,
    grid=(GRID,),
    in_specs=[
        pl.BlockSpec((NW, BN), lambda i: (0, i)),
        pl.BlockSpec((NC, BN, D), lambda i: (0, i, 0)),
        pl.BlockSpec((BN, D), lambda i: (i, 0)),
    ],
    out_specs=pl.BlockSpec((BN, D), lambda i: (i, 0)),
    out_shape=jax.ShapeDtypeStruct((NP, D), jnp.float32),
)


def _pad_edges(v):
    vt = v.reshape(NW, E_TILE)
    padv = N + (jnp.arange(E_TILE_P - E_TILE, dtype=jnp.int32) % (NP - N))
    padv = jnp.broadcast_to(padv, (NW, E_TILE_P - E_TILE))
    return jnp.concatenate([vt, padv], axis=1).reshape(NW, NCHUNK, CHUNK)


def kernel(x, edge_index, W1, b1, W2, b2):
    row = _pad_edges(edge_index[0])
    col = _pad_edges(edge_index[1])
    rc = jnp.stack([row, col], axis=2).reshape(NW * NCHUNK, 2, CHUNK)
    xp = jnp.pad(x, ((0, NP - N), (0, 0)))
    zeros = jnp.zeros((NP, D), jnp.float32)
    hist = _deg_sc(row, jnp.zeros((NP,), jnp.float32))
    h1p = _pre_tc(hist, xp, W1, b1.reshape(1, D))
    p1 = _scatter_sc(h1p, rc, zeros)
    h2p = _mid_tc(hist, p1, h1p, W2, b2.reshape(1, D))
    p2 = _scatter_sc(h2p, rc, zeros)
    return _post_tc(hist, p2, h2p)[:N]


# speedup vs baseline: 1.1708x; 1.0014x over previous
import functools

import jax
import jax.numpy as jnp
from jax import lax
from jax.experimental import pallas as pl
from jax.experimental.pallas import tpu as pltpu
from jax.experimental.pallas import tpu_sc as plsc

N = 10000
NP = 10240
E = 320000
D = 128

NC = 2
NS = 16
NW = NC * NS

E_TILE = E // NW
CHUNK = 128
NCHUNK = 79
E_TILE_P = NCHUNK * CHUNK
N_PAIR = NCHUNK // 2

ROWS_SUB = NP // NS

_mesh = plsc.VectorSubcoreMesh(core_axis_name="c", subcore_axis_name="s")


@functools.partial(
    pl.kernel,
    out_type=jax.ShapeDtypeStruct((NW, NP), jnp.float32),
    mesh=_mesh,
    compiler_params=pltpu.CompilerParams(needs_layout_passes=False,
                                         skip_device_barrier=True),
    scratch_types=[
        pltpu.VMEM((NCHUNK, CHUNK), jnp.int32),
        pltpu.VMEM((NP,), jnp.float32),
        pltpu.SemaphoreType.DMA,
        pltpu.SemaphoreType.DMA,
    ],
)
def _deg_sc(row_hbm, zeros_hbm, out_hbm, idx_all, hist_v, isem, zsem):
    c = lax.axis_index("c")
    s = lax.axis_index("s")
    wid = s * NC + c

    ild = pltpu.async_copy(row_hbm.at[wid], idx_all, isem)
    zld = pltpu.async_copy(zeros_hbm, hist_v, zsem)
    ild.wait()
    zld.wait()

    ones16 = jnp.ones((16,), jnp.float32)

    def cbody(j, carry):
        for k in range(CHUNK // 16):
            idx16 = idx_all[j, pl.ds(k * 16, 16)]
            plsc.addupdate_scatter(hist_v, [idx16], ones16)
        return carry

    lax.fori_loop(0, NCHUNK, cbody, 0)

    pltpu.sync_copy(hist_v, out_hbm.at[wid])


N_QUAD = 19


@functools.partial(
    pl.kernel,
    out_type=jax.ShapeDtypeStruct((NC, NP, D), jnp.float32),
    mesh=_mesh,
    compiler_params=pltpu.CompilerParams(skip_device_barrier=True),
    scratch_types=[
        pltpu.VMEM_SHARED((NP, D), jnp.float32),
        pltpu.VMEM((2, CHUNK), jnp.int32),
        pltpu.VMEM((2, CHUNK), jnp.int32),
        pltpu.VMEM((2, CHUNK), jnp.int32),
        pltpu.VMEM((2, CHUNK), jnp.int32),
        pltpu.VMEM((CHUNK, D), jnp.float32),
        pltpu.VMEM((CHUNK, D), jnp.float32),
        pltpu.SemaphoreType.DMA,
        pltpu.SemaphoreType.DMA,
        pltpu.SemaphoreType.DMA,
        pltpu.SemaphoreType.DMA,
        pltpu.SemaphoreType.DMA,
        pltpu.SemaphoreType.DMA,
        pltpu.SemaphoreType.DMA,
        pltpu.SemaphoreType.DMA,
        pltpu.SemaphoreType.DMA,
    ],
)
def _scatter_sc(hp_hbm, rc_hbm, zeros_hbm, out_hbm,
                acc_sh, rc0, rc1, rc2, rc3, msg0, msg1,
                gsem0, gsem1, ssem0, ssem1,
                isem0, isem1, isem2, isem3, lsem):
    c = lax.axis_index("c")
    s = lax.axis_index("s")
    wid = s * NC + c
    r0 = s * ROWS_SUB
    cbase = wid * NCHUNK

    rcs = [rc0, rc1, rc2, rc3]
    isems = [isem0, isem1, isem2, isem3]
    msgs = [msg0, msg1]
    gsems = [gsem0, gsem1]
    ssems = [ssem0, ssem1]

    def idx_load(j, k):
        pltpu.async_copy(rc_hbm.at[cbase + j], rcs[k], isems[k])

    def idx_wait(j, k):
        pltpu.make_async_copy(rc_hbm.at[cbase + j], rcs[k], isems[k]).wait()

    def gather(k, m):
        pltpu.async_copy(hp_hbm.at[rcs[k].at[0]], msgs[m], gsems[m])

    def gather_wait(k, m):
        pltpu.make_async_copy(hp_hbm.at[rcs[k].at[0]], msgs[m], gsems[m]).wait()

    def scatter(k, m):
        pltpu.async_copy(msgs[m], acc_sh.at[rcs[k].at[1]], ssems[m], add=True)

    def scatter_wait(k, m):
        pltpu.make_async_copy(msgs[m], acc_sh.at[rcs[k].at[1]], ssems[m]).wait()

    def step(cc, phase, prefetch=True):
        k, m = phase % 4, phase % 2
        if prefetch:
            idx_wait(cc + 1, (phase + 1) % 4)
            gather((phase + 1) % 4, 1 - m)
        gather_wait(k, m)
        pltpu.sync_copy(msgs[m], acc_sh.at[rcs[k].at[1]], add=True)
        if prefetch:
            idx_load(cc + 3, (phase + 3) % 4)

    zld = pltpu.async_copy(zeros_hbm.at[pl.ds(r0, ROWS_SUB)],
                           acc_sh.at[pl.ds(r0, ROWS_SUB)], lsem)
    idx_load(0, 0)
    idx_load(1, 1)
    idx_load(2, 2)
    idx_wait(0, 0)
    gather(0, 0)
    zld.wait()
    plsc.subcore_barrier()

    def quad_body(q, carry):
        base = 4 * q
        for sph in range(4):
            step(base + sph, sph)
        return carry

    lax.fori_loop(0, N_QUAD, quad_body, 0)

    idx_wait(NCHUNK - 2, 1)
    gather(1, 1)
    gather_wait(0, 0)
    pltpu.sync_copy(msg0, acc_sh.at[rc0.at[1]], add=True)
    idx_wait(NCHUNK - 1, 2)
    gather(2, 0)
    gather_wait(1, 1)
    pltpu.sync_copy(msg1, acc_sh.at[rc1.at[1]], add=True)
    gather_wait(2, 0)
    pltpu.sync_copy(msg0, acc_sh.at[rc2.at[1]], add=True)

    plsc.subcore_barrier()
    pltpu.sync_copy(acc_sh.at[pl.ds(r0, ROWS_SUB)], out_hbm.at[c, pl.ds(r0, ROWS_SUB)])


BN = 2048
GRID = NP // BN


def _dis_from_hist(hist_blk):
    deg = jnp.sum(hist_blk, axis=0) + 1.0
    return lax.rsqrt(deg)


def _pre_body(hist_ref, x_ref, w1_ref, b1_ref, out_ref):
    dis = _dis_from_hist(hist_ref[...])
    h = lax.dot_general(x_ref[...], w1_ref[...], (((1,), (1,)), ((), ())),
                        preferred_element_type=jnp.float32) + b1_ref[...]
    out_ref[...] = dis[:, None] * h


_pre_tc = pl.pallas_call(
    _pre_body,
    grid=(GRID,),
    in_specs=[
        pl.BlockSpec((NW, BN), lambda i: (0, i)),
        pl.BlockSpec((BN, D), lambda i: (i, 0)),
        pl.BlockSpec((D, D), lambda i: (0, 0)),
        pl.BlockSpec((1, D), lambda i: (0, 0)),
    ],
    out_specs=pl.BlockSpec((BN, D), lambda i: (i, 0)),
    out_shape=jax.ShapeDtypeStruct((NP, D), jnp.float32),
)


def _mid_body(hist_ref, p_ref, hp_ref, w2_ref, b2_ref, out_ref):
    dis = _dis_from_hist(hist_ref[...])[:, None]
    sacc = p_ref[0] + p_ref[1] + hp_ref[...]
    y = jnp.maximum(dis * sacc, 0.0)
    h2 = lax.dot_general(y, w2_ref[...], (((1,), (1,)), ((), ())),
                         preferred_element_type=jnp.float32) + b2_ref[...]
    out_ref[...] = dis * h2


_mid_tc = pl.pallas_call(
    _mid_body,
    grid=(GRID,),
    in_specs=[
        pl.BlockSpec((NW, BN), lambda i: (0, i)),
        pl.BlockSpec((NC, BN, D), lambda i: (0, i, 0)),
        pl.BlockSpec((BN, D), lambda i: (i, 0)),
        pl.BlockSpec((D, D), lambda i: (0, 0)),
        pl.BlockSpec((1, D), lambda i: (0, 0)),
    ],
    out_specs=pl.BlockSpec((BN, D), lambda i: (i, 0)),
    out_shape=jax.ShapeDtypeStruct((NP, D), jnp.float32),
)


def _post_body(hist_ref, p_ref, hp_ref, out_ref):
    dis = _dis_from_hist(hist_ref[...])[:, None]
    o = dis * (p_ref[0] + p_ref[1] + hp_ref[...])
    m = jnp.max(o, axis=1, keepdims=True)
    e = o - m
    out_ref[...] = e - jnp.log(jnp.sum(jnp.exp(e), axis=1, keepdims=True))


_post_tc = pl.pallas_call(
    _post_body,
    grid=(GRID,),
    in_specs=[
        pl.BlockSpec((NW, BN), lambda i: (0, i)),
        pl.BlockSpec((NC, BN, D), lambda i: (0, i, 0)),
        pl.BlockSpec((BN, D), lambda i: (i, 0)),
    ],
    out_specs=pl.BlockSpec((BN, D), lambda i: (i, 0)),
    out_shape=jax.ShapeDtypeStruct((NP, D), jnp.float32),
)


def _pad_edges(v):
    vt = v.reshape(NW, E_TILE)
    padv = N + (jnp.arange(E_TILE_P - E_TILE, dtype=jnp.int32) % (NP - N))
    padv = jnp.broadcast_to(padv, (NW, E_TILE_P - E_TILE))
    return jnp.concatenate([vt, padv], axis=1).reshape(NW, NCHUNK, CHUNK)


def kernel(x, edge_index, W1, b1, W2, b2):
    row = _pad_edges(edge_index[0])
    col = _pad_edges(edge_index[1])
    rc = jnp.stack([row, col], axis=2).reshape(NW * NCHUNK, 2, CHUNK)
    xp = jnp.pad(x, ((0, NP - N), (0, 0)))
    zeros = jnp.zeros((NP, D), jnp.float32)
    hist = _deg_sc(row, jnp.zeros((NP,), jnp.float32))
    h1p = _pre_tc(hist, xp, W1, b1.reshape(1, D))
    p1 = _scatter_sc(h1p, rc, zeros)
    h2p = _mid_tc(hist, p1, h1p, W2, b2.reshape(1, D))
    p2 = _scatter_sc(h2p, rc, zeros)
    return _post_tc(hist, p2, h2p)[:N]
